# Initial kernel scaffold; baseline (speedup 1.0000x reference)
import jax
import jax.numpy as jnp
from jax.experimental import pallas as pl


def kernel(x, edge_index, W_w, W_b, a_w, a_b):
    raise NotImplementedError("write your pallas kernel here")


# speedup vs baseline: 5.8367x; 5.8367x over previous
import jax
import jax.numpy as jnp
from jax import lax
from jax.experimental import pallas as pl
from jax.experimental.pallas import tpu as pltpu
from jax.experimental.pallas import tpu_sc as plsc

N_NODES = 10000
N_EDGES = 320000
DIM = 128
NC = 2
NS = 16
NW = NC * NS
EPT = N_EDGES // NW
K = 80
NCH = EPT // K
RPT = N_NODES // NS
L = 16
NV = DIM // L
ROWBLK = 125
ZBLK = 25


def _tc_h_body(x_ref, wt_ref, b_ref, a2_ref, h_ref, al_ref):
    h = jnp.dot(x_ref[...], wt_ref[...], preferred_element_type=jnp.float32)
    h = h + b_ref[...]
    h_ref[...] = h
    al_ref[...] = jnp.dot(h, a2_ref[...], preferred_element_type=jnp.float32)


def _edge_logits_body(as_hbm, ad_hbm, row_hbm, col_hbm, par_hbm,
                      e_hbm, ms_hbm,
                      as_v, ad_v, row_v, col_v, e_v, par_v, ms_v):
    cid = lax.axis_index("c")
    sid = lax.axis_index("s")
    wid = sid * NC + cid
    pltpu.sync_copy(as_hbm, as_v)
    pltpu.sync_copy(ad_hbm, ad_v)
    pltpu.sync_copy(row_hbm.at[wid], row_v)
    pltpu.sync_copy(col_hbm.at[wid], col_v)
    pltpu.sync_copy(par_hbm, par_v)
    pv = par_v[...]
    ab = pv[0]

    def pass1(j, m_run):
        for k in range(K // L):
            sl = pl.ds(k * L, L)
            r = row_v[j, sl]
            c = col_v[j, sl]
            z = plsc.load_gather(as_v, [r]) + plsc.load_gather(ad_v, [c]) + ab
            e = jnp.where(z >= 0.0, z, z * 0.01)
            e_v[j, sl] = e
            m_run = jnp.maximum(m_run, e)
        return m_run

    m_run = lax.fori_loop(0, NCH, pass1, jnp.full((L,), -1e30, jnp.float32))
    m_t = jnp.max(m_run)

    def pass2(j, s_run):
        for k in range(K // L):
            s_run = s_run + jnp.exp(e_v[j, pl.ds(k * L, L)] - m_t)
        return s_run

    s_run = lax.fori_loop(0, NCH, pass2, jnp.zeros((L,), jnp.float32))
    s_t = jnp.sum(s_run)
    pltpu.sync_copy(e_v, e_hbm.at[wid])
    ii = lax.iota(jnp.int32, L)
    ms_v[...] = jnp.where(ii == 0, m_t, jnp.where(ii == 1, s_t, 0.0))
    pltpu.sync_copy(ms_v, ms_hbm.at[wid])


def _gat_scatter_body(h_hbm, row_hbm, col_hbm, e_hbm, par_hbm,
                      out_hbm,
                      rowb, colb, e_v, rows_v, zbuf, par_v, acc):
    cid = lax.axis_index("c")
    sid = lax.axis_index("s")
    wid = sid * NC + cid
    pltpu.sync_copy(e_hbm.at[wid], e_v)
    pltpu.sync_copy(par_hbm, par_v)
    pv = par_v[...]
    m_g = pv[0]
    inv_s = pv[1]

    def to_att(j, carry):
        for k in range(K // L):
            sl = pl.ds(k * L, L)
            e_v[j, sl] = jnp.exp(e_v[j, sl] - m_g) * inv_s
        return carry

    lax.fori_loop(0, NCH, to_att, 0)

    z16 = jnp.zeros((L,), jnp.float32)

    def zero_row(j, carry):
        for v in range(NV):
            zbuf[j, pl.ds(v * L, L)] = z16
        return carry

    lax.fori_loop(0, ZBLK, zero_row, 0)
    for t in range(RPT // ZBLK):
        pltpu.sync_copy(zbuf, acc.at[pl.ds(sid * RPT + t * ZBLK, ZBLK)])
    plsc.subcore_barrier()

    def chunk(j, carry):
        pltpu.sync_copy(col_hbm.at[wid, j], colb)
        pltpu.sync_copy(row_hbm.at[wid, j], rowb)
        pltpu.sync_copy(h_hbm.at[colb.at[0]], rows_v)
        j16 = jnp.full((L,), j, jnp.int32)

        def scale(i, c2):
            a = plsc.load_gather(e_v, [j16, jnp.full((L,), i, jnp.int32)])
            for v in range(NV):
                s2 = pl.ds(v * L, L)
                rows_v[i, s2] = rows_v[i, s2] * a
            return c2

        lax.fori_loop(0, K, scale, 0)
        pltpu.sync_copy(rows_v, acc.at[rowb.at[0]], add=True)
        return carry

    lax.fori_loop(0, NCH, chunk, 0)
    plsc.subcore_barrier()
    for t in range(RPT // ROWBLK):
        sl = pl.ds(sid * RPT + t * ROWBLK, ROWBLK)
        pltpu.sync_copy(acc.at[sl], out_hbm.at[cid, sid, t])


_SC_KERNEL_CACHE = []


def _make_sc_kernels():
    if _SC_KERNEL_CACHE:
        return _SC_KERNEL_CACHE[0]
    mesh = plsc.VectorSubcoreMesh(core_axis_name="c", subcore_axis_name="s",
                                  num_cores=NC, num_subcores=NS)
    edge_logits = pl.kernel(
        _edge_logits_body,
        out_type=(jax.ShapeDtypeStruct((NW, NCH, K), jnp.float32),
                  jax.ShapeDtypeStruct((NW, L), jnp.float32)),
        mesh=mesh,
        compiler_params=pltpu.CompilerParams(needs_layout_passes=False),
        scratch_types=[
            pltpu.VMEM((N_NODES,), jnp.float32),
            pltpu.VMEM((N_NODES,), jnp.float32),
            pltpu.VMEM((NCH, K), jnp.int32),
            pltpu.VMEM((NCH, K), jnp.int32),
            pltpu.VMEM((NCH, K), jnp.float32),
            pltpu.VMEM((L,), jnp.float32),
            pltpu.VMEM((L,), jnp.float32),
        ],
    )
    gat_scatter = pl.kernel(
        _gat_scatter_body,
        out_type=jax.ShapeDtypeStruct(
            (NC, NS, RPT // ROWBLK, ROWBLK, DIM), jnp.float32),
        mesh=mesh,
        compiler_params=pltpu.CompilerParams(needs_layout_passes=False),
        scratch_types=[
            pltpu.VMEM((1, K), jnp.int32),
            pltpu.VMEM((1, K), jnp.int32),
            pltpu.VMEM((NCH, K), jnp.float32),
            pltpu.VMEM((K, DIM), jnp.float32),
            pltpu.VMEM((ZBLK, DIM), jnp.float32),
            pltpu.VMEM((L,), jnp.float32),
            pltpu.VMEM_SHARED((N_NODES, DIM), jnp.float32),
        ],
    )
    _SC_KERNEL_CACHE.append((edge_logits, gat_scatter))
    return edge_logits, gat_scatter


_TC_BLK = 1000


def kernel(x, edge_index, W_w, W_b, a_w, a_b):
    wt = W_w.T
    b2 = W_b.reshape(1, DIM)
    a2 = jnp.zeros((DIM, DIM), jnp.float32)
    a2 = a2.at[:, 0].set(a_w[0, :DIM]).at[:, 1].set(a_w[0, DIM:])
    h, al = pl.pallas_call(
        _tc_h_body,
        grid=(N_NODES // _TC_BLK,),
        in_specs=[
            pl.BlockSpec((_TC_BLK, DIM), lambda i: (i, 0)),
            pl.BlockSpec((DIM, DIM), lambda i: (0, 0)),
            pl.BlockSpec((1, DIM), lambda i: (0, 0)),
            pl.BlockSpec((DIM, DIM), lambda i: (0, 0)),
        ],
        out_specs=[
            pl.BlockSpec((_TC_BLK, DIM), lambda i: (i, 0)),
            pl.BlockSpec((_TC_BLK, DIM), lambda i: (i, 0)),
        ],
        out_shape=[
            jax.ShapeDtypeStruct((N_NODES, DIM), jnp.float32),
            jax.ShapeDtypeStruct((N_NODES, DIM), jnp.float32),
        ],
    )(x, wt, b2, a2)
    alpha_s = al[:, 0]
    alpha_d = al[:, 1]
    row3 = edge_index[0].reshape(NW, NCH, K)
    col3 = edge_index[1].reshape(NW, NCH, K)
    row4 = edge_index[0].reshape(NW, NCH, 1, K)
    col4 = edge_index[1].reshape(NW, NCH, 1, K)
    par1 = jnp.zeros((L,), jnp.float32).at[0].set(a_b[0])
    edge_logits, gat_scatter = _make_sc_kernels()
    e3, ms = edge_logits(alpha_s, alpha_d, row3, col3, par1)
    m = ms[:, 0]
    s = ms[:, 1]
    m_g = jnp.max(m)
    s_g = jnp.sum(s * jnp.exp(m - m_g))
    par2 = jnp.zeros((L,), jnp.float32).at[0].set(m_g).at[1].set(1.0 / s_g)
    partials = gat_scatter(h, row4, col4, e3, par2)
    partials = partials.reshape(NC, N_NODES, DIM)
    return partials[0] + partials[1]


# speedup vs baseline: 5.9732x; 1.0234x over previous
import jax
import jax.numpy as jnp
from jax import lax
from jax.experimental import pallas as pl
from jax.experimental.pallas import tpu as pltpu
from jax.experimental.pallas import tpu_sc as plsc

N_NODES = 10000
N_EDGES = 320000
DIM = 128
NC = 2
NS = 16
NW = NC * NS
EPT = N_EDGES // NW
K = 80
NCH = EPT // K
NCH_P = 127
RPT = N_NODES // NS
L = 16
NV = DIM // L
ROWBLK = 125
ZBLK = 25


def _tc_h_body(x_ref, wt_ref, b_ref, a2_ref, h_ref, al_ref):
    h = jnp.dot(x_ref[...], wt_ref[...], preferred_element_type=jnp.float32)
    h = h + b_ref[...]
    h_ref[...] = h
    al_ref[...] = jnp.dot(h, a2_ref[...], preferred_element_type=jnp.float32)


def _edge_logits_body(as_hbm, ad_hbm, row_hbm, col_hbm, par_hbm,
                      e_hbm, ms_hbm,
                      as_v, ad_v, row_v, col_v, e_v, par_v, ms_v):
    cid = lax.axis_index("c")
    sid = lax.axis_index("s")
    wid = sid * NC + cid
    pltpu.sync_copy(as_hbm, as_v)
    pltpu.sync_copy(ad_hbm, ad_v)
    pltpu.sync_copy(row_hbm.at[wid], row_v)
    pltpu.sync_copy(col_hbm.at[wid], col_v)
    pltpu.sync_copy(par_hbm, par_v)
    pv = par_v[...]
    ab = pv[0]

    def pass1(j, m_run):
        for k in range(K // L):
            sl = pl.ds(k * L, L)
            r = row_v[j, sl]
            c = col_v[j, sl]
            z = plsc.load_gather(as_v, [r]) + plsc.load_gather(ad_v, [c]) + ab
            e = jnp.where(z >= 0.0, z, z * 0.01)
            e_v[j, sl] = e
            m_run = jnp.maximum(m_run, e)
        return m_run

    m_run = lax.fori_loop(0, NCH, pass1, jnp.full((L,), -1e30, jnp.float32))
    m_t = jnp.max(m_run)

    def pass2(j, s_run):
        for k in range(K // L):
            s_run = s_run + jnp.exp(e_v[j, pl.ds(k * L, L)] - m_t)
        return s_run

    s_run = lax.fori_loop(0, NCH, pass2, jnp.zeros((L,), jnp.float32))
    s_t = jnp.sum(s_run)
    pltpu.sync_copy(e_v, e_hbm.at[wid])
    ii = lax.iota(jnp.int32, L)
    ms_v[...] = jnp.where(ii == 0, m_t, jnp.where(ii == 1, s_t, 0.0))
    pltpu.sync_copy(ms_v, ms_hbm.at[wid])


def _gat_scatter_body(h_hbm, row_hbm, col_hbm, e_hbm, par_hbm,
                      out_hbm,
                      rows0, rows1, colb0, colb1, rowb0, rowb1,
                      eb0, eb1, attb0, attb1, zbuf, par_v, acc,
                      isem0, isem1, gsem0, gsem1, ssem0, ssem1):
    cid = lax.axis_index("c")
    sid = lax.axis_index("s")
    wid = sid * NC + cid
    pltpu.sync_copy(par_hbm, par_v)
    pv = par_v[...]
    m_g = pv[0]
    inv_s = pv[1]

    z16 = jnp.zeros((L,), jnp.float32)

    def zero_row(j, carry):
        for v in range(NV):
            zbuf[j, pl.ds(v * L, L)] = z16
        return carry

    lax.fori_loop(0, ZBLK, zero_row, 0)
    for t in range(RPT // ZBLK):
        pltpu.sync_copy(zbuf, acc.at[pl.ds(sid * RPT + t * ZBLK, ZBLK)])
    plsc.subcore_barrier()

    def issue_idx(j, colb, rowb, eb, isem):
        pltpu.async_copy(col_hbm.at[wid, j], colb, isem)
        pltpu.async_copy(row_hbm.at[wid, j], rowb, isem)
        pltpu.async_copy(e_hbm.at[wid, j], eb, isem)

    def wait_idx(j, colb, rowb, eb, isem):
        pltpu.make_async_copy(col_hbm.at[wid, j], colb, isem).wait()
        pltpu.make_async_copy(row_hbm.at[wid, j], rowb, isem).wait()
        pltpu.make_async_copy(e_hbm.at[wid, j], eb, isem).wait()

    def to_att(eb, attb):
        for k in range(K // L):
            sl = pl.ds(k * L, L)
            attb[sl] = jnp.exp(eb[0, sl] - m_g) * inv_s

    def scale(rows, attb):
        def body(i, c2):
            a = plsc.load_gather(attb, [jnp.full((L,), i, jnp.int32)])
            for v in range(NV):
                s2 = pl.ds(v * L, L)
                rows[i, s2] = rows[i, s2] * a
            return c2

        lax.fori_loop(0, K, body, 0)

    issue_idx(0, colb0, rowb0, eb0, isem0)
    wait_idx(0, colb0, rowb0, eb0, isem0)
    pltpu.async_copy(h_hbm.at[colb0.at[0]], rows0, gsem0)
    issue_idx(1, colb1, rowb1, eb1, isem1)

    def pair(t, carry):
        a = 2 * t + 1
        b = 2 * t + 2
        nxt = jnp.minimum(a + 2, NCH_P - 1)
        wait_idx(a, colb1, rowb1, eb1, isem1)
        pltpu.async_copy(h_hbm.at[colb1.at[0]], rows1, gsem1)
        pltpu.make_async_copy(h_hbm.at[colb0.at[0]], rows0, gsem0).wait()
        to_att(eb0, attb0)
        scale(rows0, attb0)
        pltpu.async_copy(rows0, acc.at[rowb0.at[0]], ssem0, add=True)
        pltpu.make_async_copy(h_hbm.at[colb1.at[0]], rows1, gsem1).wait()
        to_att(eb1, attb1)
        pltpu.make_async_copy(rows0, acc.at[rowb0.at[0]], ssem0).wait()
        issue_idx(b, colb0, rowb0, eb0, isem0)
        scale(rows1, attb1)
        pltpu.async_copy(rows1, acc.at[rowb1.at[0]], ssem1, add=True)
        wait_idx(b, colb0, rowb0, eb0, isem0)
        pltpu.async_copy(h_hbm.at[colb0.at[0]], rows0, gsem0)
        pltpu.make_async_copy(rows1, acc.at[rowb1.at[0]], ssem1).wait()
        issue_idx(nxt, colb1, rowb1, eb1, isem1)
        return carry

    lax.fori_loop(0, (NCH_P - 1) // 2, pair, 0)
    wait_idx(NCH_P - 1, colb1, rowb1, eb1, isem1)
    pltpu.make_async_copy(h_hbm.at[colb0.at[0]], rows0, gsem0).wait()
    to_att(eb0, attb0)
    scale(rows0, attb0)
    pltpu.async_copy(rows0, acc.at[rowb0.at[0]], ssem0, add=True)
    pltpu.make_async_copy(rows0, acc.at[rowb0.at[0]], ssem0).wait()
    plsc.subcore_barrier()
    for t in range(RPT // ROWBLK):
        sl = pl.ds(sid * RPT + t * ROWBLK, ROWBLK)
        pltpu.sync_copy(acc.at[sl], out_hbm.at[cid, sid, t])


_SC_KERNEL_CACHE = []


def _make_sc_kernels():
    if _SC_KERNEL_CACHE:
        return _SC_KERNEL_CACHE[0]
    mesh = plsc.VectorSubcoreMesh(core_axis_name="c", subcore_axis_name="s",
                                  num_cores=NC, num_subcores=NS)
    edge_logits = pl.kernel(
        _edge_logits_body,
        out_type=(jax.ShapeDtypeStruct((NW, NCH, K), jnp.float32),
                  jax.ShapeDtypeStruct((NW, L), jnp.float32)),
        mesh=mesh,
        compiler_params=pltpu.CompilerParams(needs_layout_passes=False),
        scratch_types=[
            pltpu.VMEM((N_NODES,), jnp.float32),
            pltpu.VMEM((N_NODES,), jnp.float32),
            pltpu.VMEM((NCH, K), jnp.int32),
            pltpu.VMEM((NCH, K), jnp.int32),
            pltpu.VMEM((NCH, K), jnp.float32),
            pltpu.VMEM((L,), jnp.float32),
            pltpu.VMEM((L,), jnp.float32),
        ],
    )
    gat_scatter = pl.kernel(
        _gat_scatter_body,
        out_type=jax.ShapeDtypeStruct(
            (NC, NS, RPT // ROWBLK, ROWBLK, DIM), jnp.float32),
        mesh=mesh,
        compiler_params=pltpu.CompilerParams(needs_layout_passes=False),
        scratch_types=[
            pltpu.VMEM((K, DIM), jnp.float32),
            pltpu.VMEM((K, DIM), jnp.float32),
            pltpu.VMEM((1, K), jnp.int32),
            pltpu.VMEM((1, K), jnp.int32),
            pltpu.VMEM((1, K), jnp.int32),
            pltpu.VMEM((1, K), jnp.int32),
            pltpu.VMEM((1, K), jnp.float32),
            pltpu.VMEM((1, K), jnp.float32),
            pltpu.VMEM((K,), jnp.float32),
            pltpu.VMEM((K,), jnp.float32),
            pltpu.VMEM((ZBLK, DIM), jnp.float32),
            pltpu.VMEM((L,), jnp.float32),
            pltpu.VMEM_SHARED((N_NODES, DIM), jnp.float32),
            pltpu.SemaphoreType.DMA,
            pltpu.SemaphoreType.DMA,
            pltpu.SemaphoreType.DMA,
            pltpu.SemaphoreType.DMA,
            pltpu.SemaphoreType.DMA,
            pltpu.SemaphoreType.DMA,
        ],
    )
    _SC_KERNEL_CACHE.append((edge_logits, gat_scatter))
    return edge_logits, gat_scatter


_TC_BLK = 1000


def kernel(x, edge_index, W_w, W_b, a_w, a_b):
    wt = W_w.T
    b2 = W_b.reshape(1, DIM)
    a2 = jnp.zeros((DIM, DIM), jnp.float32)
    a2 = a2.at[:, 0].set(a_w[0, :DIM]).at[:, 1].set(a_w[0, DIM:])
    h, al = pl.pallas_call(
        _tc_h_body,
        grid=(N_NODES // _TC_BLK,),
        in_specs=[
            pl.BlockSpec((_TC_BLK, DIM), lambda i: (i, 0)),
            pl.BlockSpec((DIM, DIM), lambda i: (0, 0)),
            pl.BlockSpec((1, DIM), lambda i: (0, 0)),
            pl.BlockSpec((DIM, DIM), lambda i: (0, 0)),
        ],
        out_specs=[
            pl.BlockSpec((_TC_BLK, DIM), lambda i: (i, 0)),
            pl.BlockSpec((_TC_BLK, DIM), lambda i: (i, 0)),
        ],
        out_shape=[
            jax.ShapeDtypeStruct((N_NODES, DIM), jnp.float32),
            jax.ShapeDtypeStruct((N_NODES, DIM), jnp.float32),
        ],
    )(x, wt, b2, a2)
    alpha_s = al[:, 0]
    alpha_d = al[:, 1]
    row3 = edge_index[0].reshape(NW, NCH, K)
    col3 = edge_index[1].reshape(NW, NCH, K)
    zpad = jnp.zeros((NW, NCH_P - NCH, 1, K), jnp.int32)
    row4 = jnp.concatenate(
        [edge_index[0].reshape(NW, NCH, 1, K), zpad], axis=1)
    col4 = jnp.concatenate(
        [edge_index[1].reshape(NW, NCH, 1, K), zpad], axis=1)
    par1 = jnp.zeros((L,), jnp.float32).at[0].set(a_b[0])
    edge_logits, gat_scatter = _make_sc_kernels()
    e3, ms = edge_logits(alpha_s, alpha_d, row3, col3, par1)
    m = ms[:, 0]
    s = ms[:, 1]
    m_g = jnp.max(m)
    s_g = jnp.sum(s * jnp.exp(m - m_g))
    par2 = jnp.zeros((L,), jnp.float32).at[0].set(m_g).at[1].set(1.0 / s_g)
    e4 = jnp.concatenate(
        [e3.reshape(NW, NCH, 1, K),
         jnp.full((NW, NCH_P - NCH, 1, K), -1e30, jnp.float32)], axis=1)
    partials = gat_scatter(h, row4, col4, e4, par2)
    partials = partials.reshape(NC, N_NODES, DIM)
    return partials[0] + partials[1]
